# Initial kernel scaffold; baseline (speedup 1.0000x reference)
import jax
import jax.numpy as jnp
from jax.experimental import pallas as pl


def kernel(f_atoms, f_bonds, w_bonds, a2b, b2a, b2revb, batch, W_i, W_h, W_o, b_o):
    raise NotImplementedError("write your pallas kernel here")


# speedup vs baseline: 1.4128x; 1.4128x over previous
import functools

import jax
import jax.numpy as jnp
from jax import lax
from jax.experimental import pallas as pl
from jax.experimental.pallas import tpu as pltpu
from jax.experimental.pallas import tpu_sc as plsc

NC = 2
NS = 16
NW = NC * NS
L = 16
H = 128
NM = 256


def _mesh():
    return plsc.VectorSubcoreMesh(
        core_axis_name="c", subcore_axis_name="s", num_cores=NC, num_subcores=NS
    )


_SC_PARAMS = pltpu.CompilerParams(needs_layout_passes=False)


def _wid():
    return lax.axis_index("s") * NC + lax.axis_index("c")


def _full16(i):
    return jnp.full((L,), i, dtype=jnp.int32)


def _sc_wgather(w_bonds, idx2d):
    R, C = idx2d.shape
    rpw = R // NW

    @functools.partial(
        pl.kernel,
        mesh=_mesh(),
        compiler_params=_SC_PARAMS,
        out_type=jax.ShapeDtypeStruct((R, C), jnp.float32),
        scratch_types=[
            pltpu.VMEM((rpw, C), jnp.int32),
            pltpu.VMEM((rpw, C), jnp.float32),
            pltpu.SemaphoreType.DMA,
        ],
    )
    def k(w_hbm, idx_hbm, out_hbm, idx_v, out_v, sem):
        wid = _wid()
        base = wid * rpw
        pltpu.sync_copy(idx_hbm.at[pl.ds(base, rpw)], idx_v)

        def body(c, carry):
            pltpu.async_copy(w_hbm.at[idx_v.at[c]], out_v.at[c], sem).wait()
            return carry

        lax.fori_loop(0, rpw, body, 0)
        pltpu.sync_copy(out_v, out_hbm.at[pl.ds(base, rpw)])

    return k(w_bonds, idx2d)


def _sc_atom_gather(msg, w1d, idx1d, apply_relu, ch=256):
    TOT = idx1d.shape[0]
    AP = TOT // 32
    wpw = TOT // NW
    CH = ch
    G = CH // 32
    rpw = wpw // CH
    apw = rpw * G

    @functools.partial(
        pl.kernel,
        mesh=_mesh(),
        compiler_params=_SC_PARAMS,
        out_type=jax.ShapeDtypeStruct((AP, H), jnp.float32),
        scratch_types=[
            pltpu.VMEM((wpw,), jnp.int32),
            pltpu.VMEM((wpw,), jnp.float32),
            pltpu.VMEM((2, CH, H), jnp.float32),
            pltpu.VMEM((apw, H), jnp.float32),
            pltpu.SemaphoreType.DMA,
        ],
    )
    def k(msg_hbm, w_hbm, idx_hbm, out_hbm, idx_v, w_v, rows_v, out_v, sem):
        wid = _wid()
        pltpu.sync_copy(idx_hbm.at[pl.ds(wid * wpw, wpw)], idx_v)
        pltpu.sync_copy(w_hbm.at[pl.ds(wid * wpw, wpw)], w_v)

        pltpu.async_copy(
            msg_hbm.at[idx_v.at[pl.ds(0, CH)]], rows_v.at[0], sem)

        def do_chunk(c, b):
            @pl.when(c + 1 < rpw)
            def _():
                pltpu.async_copy(
                    msg_hbm.at[idx_v.at[pl.ds((c + 1) * CH, CH)]],
                    rows_v.at[1 - b], sem)

            pltpu.make_async_copy(
                msg_hbm.at[idx_v.at[pl.ds(c * CH, CH)]],
                rows_v.at[b], sem).wait()
            rb = rows_v.at[b]
            for g in range(G):
                zeros = tuple(
                    jnp.zeros((L,), jnp.float32) for _ in range(H // L))

                @plsc.parallel_loop(0, 32, unroll=4, carry=zeros)
                def accs(kk, accs_in):
                    r = g * 32 + kk
                    wv = plsc.load_gather(w_v, [_full16(c * CH + r)])
                    new = []
                    for j in range(H // L):
                        x = rb[r, pl.ds(j * L, L)]
                        if apply_relu:
                            x = jnp.maximum(x, 0.0)
                        new.append(accs_in[j] + wv * x)
                    return tuple(new)

                for j in range(H // L):
                    out_v[c * G + g, pl.ds(j * L, L)] = accs[j]

        def pair(t, carry):
            do_chunk(2 * t, 0)
            do_chunk(2 * t + 1, 1)
            return carry

        lax.fori_loop(0, rpw // 2, pair, 0)
        pltpu.sync_copy(out_v, out_hbm.at[pl.ds(wid * apw, apw)])

    return k(msg, w1d, idx1d)


def _sc_bond_update(a_msg, msg, w_bonds, b2a, b2revb, apply_relu, kb=128):
    NB = b2a.shape[0]
    bpw = NB // NW
    K = kb
    full = bpw // K
    tail = bpw - full * K

    @functools.partial(
        pl.kernel,
        mesh=_mesh(),
        compiler_params=_SC_PARAMS,
        out_type=jax.ShapeDtypeStruct((NB, H), jnp.float32),
        scratch_types=[
            pltpu.VMEM((bpw,), jnp.int32),
            pltpu.VMEM((bpw,), jnp.int32),
            pltpu.VMEM((bpw,), jnp.float32),
            pltpu.VMEM((2, K, H), jnp.float32),
            pltpu.VMEM((2, K, H), jnp.float32),
            pltpu.VMEM((2, K, H), jnp.float32),
            pltpu.SemaphoreType.DMA,
            pltpu.SemaphoreType.DMA,
        ],
    )
    def k(am_hbm, msg_hbm, w_hbm, ba_hbm, br_hbm, out_hbm,
          idxa, idxr, w_v, am_v, rev_v, o_v, semg, semw):
        wid = _wid()
        eb = wid * bpw
        pltpu.sync_copy(ba_hbm.at[pl.ds(eb, bpw)], idxa)
        pltpu.sync_copy(br_hbm.at[pl.ds(eb, bpw)], idxr)
        pltpu.sync_copy(w_hbm.at[pl.ds(eb, bpw)], w_v)

        def _issue(c, b):
            pltpu.async_copy(
                am_hbm.at[idxa.at[pl.ds(c * K, K)]], am_v.at[b], semg)
            pltpu.async_copy(
                msg_hbm.at[idxr.at[pl.ds(c * K, K)]], rev_v.at[b], semg)

        _issue(0, 0)

        def compute(c, b, n):
            amb, revb, ob = am_v.at[b], rev_v.at[b], o_v.at[b]

            @plsc.parallel_loop(0, n, unroll=8)
            def _(r):
                wv = plsc.load_gather(w_v, [_full16(c * K + r)])
                for j in range(H // L):
                    x = revb[r, pl.ds(j * L, L)]
                    if apply_relu:
                        x = jnp.maximum(x, 0.0)
                    ob[r, pl.ds(j * L, L)] = amb[r, pl.ds(j * L, L)] - wv * x

        def do_chunk(c, b):
            @pl.when(c + 1 < full)
            def _():
                _issue(c + 1, 1 - b)

            pltpu.make_async_copy(
                am_hbm.at[idxa.at[pl.ds(c * K, K)]], am_v.at[b], semg).wait()
            pltpu.make_async_copy(
                msg_hbm.at[idxr.at[pl.ds(c * K, K)]], rev_v.at[b], semg).wait()

            @pl.when(c >= 2)
            def _():
                pltpu.make_async_copy(
                    o_v.at[b], out_hbm.at[pl.ds(eb + (c - 2) * K, K)],
                    semw).wait()

            compute(c, b, K)
            pltpu.async_copy(o_v.at[b], out_hbm.at[pl.ds(eb + c * K, K)], semw)

        def pair(t, carry):
            do_chunk(2 * t, 0)
            do_chunk(2 * t + 1, 1)
            return carry

        lax.fori_loop(0, full // 2, pair, 0)
        for t in (full - 2, full - 1):
            pltpu.make_async_copy(
                o_v.at[t % 2], out_hbm.at[pl.ds(eb + t * K, K)], semw).wait()
        if tail:
            cb = full * K
            pltpu.async_copy(
                am_hbm.at[idxa.at[pl.ds(cb, tail)]],
                am_v.at[0].at[pl.ds(0, tail)], semg)
            pltpu.async_copy(
                msg_hbm.at[idxr.at[pl.ds(cb, tail)]],
                rev_v.at[0].at[pl.ds(0, tail)], semg)
            pltpu.make_async_copy(
                am_hbm.at[idxa.at[pl.ds(cb, tail)]],
                am_v.at[0].at[pl.ds(0, tail)], semg).wait()
            pltpu.make_async_copy(
                msg_hbm.at[idxr.at[pl.ds(cb, tail)]],
                rev_v.at[0].at[pl.ds(0, tail)], semg).wait()
            compute(full, 0, tail)
            pltpu.sync_copy(
                o_v.at[0].at[pl.ds(0, tail)],
                out_hbm.at[pl.ds(eb + cb, tail)])

    return k(a_msg, msg, w_bonds, b2a, b2revb)


def _mm_init(x, W, bk=2560):
    n = x.shape[0]

    def body(x_ref, w_ref, o_ref):
        o_ref[...] = jnp.dot(x_ref[...], w_ref[...],
                             preferred_element_type=jnp.float32)

    return pl.pallas_call(
        body,
        grid=(n // bk,),
        in_specs=[
            pl.BlockSpec((bk, H), lambda i: (i, 0)),
            pl.BlockSpec((H, H), lambda i: (0, 0)),
        ],
        out_specs=pl.BlockSpec((bk, H), lambda i: (i, 0)),
        out_shape=jax.ShapeDtypeStruct((n, H), jnp.float32),
    )(x, W)


def _mm_step(pre, inp, W, bk=2560):
    n = pre.shape[0]

    def body(x_ref, inp_ref, w_ref, o_ref):
        y = jnp.dot(x_ref[...], w_ref[...], preferred_element_type=jnp.float32)
        o_ref[...] = jnp.maximum(inp_ref[...] + y, 0.0)

    return pl.pallas_call(
        body,
        grid=(n // bk,),
        in_specs=[
            pl.BlockSpec((bk, H), lambda i: (i, 0)),
            pl.BlockSpec((bk, H), lambda i: (i, 0)),
            pl.BlockSpec((H, H), lambda i: (0, 0)),
        ],
        out_specs=pl.BlockSpec((bk, H), lambda i: (i, 0)),
        out_shape=jax.ShapeDtypeStruct((n, H), jnp.float32),
    )(pre, inp, W)


def _mm_final(f_atoms_p, am_p, batch3d, Wo_a, Wo_m, b_o2d, bk=1024):
    n = f_atoms_p.shape[0]
    nblk = n // bk

    def body(fa_ref, am_ref, b_ref, woa_ref, wom_ref, bo_ref, o_ref,
             acc_ref, cnt_ref):
        i = pl.program_id(0)

        @pl.when(i == 0)
        def _():
            acc_ref[...] = jnp.zeros_like(acc_ref)
            cnt_ref[...] = jnp.zeros_like(cnt_ref)

        h = jnp.dot(fa_ref[...], woa_ref[...], preferred_element_type=jnp.float32)
        h = h + jnp.dot(am_ref[...], wom_ref[...], preferred_element_type=jnp.float32)
        h = jnp.maximum(h + bo_ref[...], 0.0)
        ids = b_ref[0, 0, :]
        mol = lax.broadcasted_iota(jnp.int32, (NM, bk), 0)
        onehot = (ids[None, :] == mol).astype(jnp.float32)
        acc_ref[...] += jnp.dot(onehot, h, preferred_element_type=jnp.float32)
        cnt_ref[...] += jnp.sum(onehot, axis=1, keepdims=True)

        @pl.when(i == nblk - 1)
        def _():
            o_ref[...] = acc_ref[...] / jnp.maximum(cnt_ref[...], 1.0)

    return pl.pallas_call(
        body,
        grid=(nblk,),
        in_specs=[
            pl.BlockSpec((bk, H), lambda i: (i, 0)),
            pl.BlockSpec((bk, H), lambda i: (i, 0)),
            pl.BlockSpec((1, 1, bk), lambda i: (i, 0, 0)),
            pl.BlockSpec((H, H), lambda i: (0, 0)),
            pl.BlockSpec((H, H), lambda i: (0, 0)),
            pl.BlockSpec((1, H), lambda i: (0, 0)),
        ],
        out_specs=pl.BlockSpec((NM, H), lambda i: (0, 0)),
        out_shape=jax.ShapeDtypeStruct((NM, H), jnp.float32),
        scratch_shapes=[
            pltpu.VMEM((NM, H), jnp.float32),
            pltpu.VMEM((NM, 1), jnp.float32),
        ],
    )(f_atoms_p, am_p, batch3d, Wo_a, Wo_m, b_o2d)


def kernel(f_atoms, f_bonds, w_bonds, a2b, b2a, b2revb, batch,
           W_i, W_h, W_o, b_o):
    n_atoms = f_atoms.shape[0]
    nb = f_bonds.shape[0]
    maxnb = a2b.shape[1]

    a2b = a2b.astype(jnp.int32)
    b2a = b2a.astype(jnp.int32)
    b2revb = b2revb.astype(jnp.int32)

    ap = -(-n_atoms // 1024) * 1024
    a2b_p = jnp.pad(a2b, ((0, ap - n_atoms), (0, 0)))
    idx2d = a2b_p.reshape(ap * maxnb // 128, 128)
    idx1d = a2b_p.reshape(-1)

    w1d = _sc_wgather(w_bonds, idx2d).reshape(-1)

    inp = _mm_init(f_bonds, W_i)

    am = _sc_atom_gather(inp, w1d, idx1d, True)
    pre = _sc_bond_update(am, inp, w_bonds, b2a, b2revb, True)
    msg = _mm_step(pre, inp, W_h)

    am = _sc_atom_gather(msg, w1d, idx1d, False)
    pre = _sc_bond_update(am, msg, w_bonds, b2a, b2revb, False)
    msg = _mm_step(pre, inp, W_h)

    am = _sc_atom_gather(msg, w1d, idx1d, False)
    f_atoms_p = jnp.pad(f_atoms, ((0, ap - n_atoms), (0, 0)))
    batch_p = jnp.pad(batch.astype(jnp.int32), (0, ap - n_atoms),
                      constant_values=-1)
    batch3d = batch_p.reshape(ap // 1024, 1, 1024)
    Wo_a = W_o[:f_atoms.shape[1], :]
    Wo_m = W_o[f_atoms.shape[1]:, :]
    b_o2d = b_o.reshape(1, H)

    return _mm_final(f_atoms_p, am, batch3d, Wo_a, Wo_m, b_o2d)


# speedup vs baseline: 1.4984x; 1.0606x over previous
import functools

import jax
import jax.numpy as jnp
from jax import lax
from jax.experimental import pallas as pl
from jax.experimental.pallas import tpu as pltpu
from jax.experimental.pallas import tpu_sc as plsc

NC = 2
NS = 16
NW = NC * NS
L = 16
H = 128
NM = 256


def _mesh():
    return plsc.VectorSubcoreMesh(
        core_axis_name="c", subcore_axis_name="s", num_cores=NC, num_subcores=NS
    )


_SC_PARAMS = pltpu.CompilerParams(needs_layout_passes=False)


def _wid():
    return lax.axis_index("s") * NC + lax.axis_index("c")


def _full16(i):
    return jnp.full((L,), i, dtype=jnp.int32)


def _sc_wgather(w_bonds, idx2d):
    R, C = idx2d.shape
    rpw = R // NW

    @functools.partial(
        pl.kernel,
        mesh=_mesh(),
        compiler_params=_SC_PARAMS,
        out_type=jax.ShapeDtypeStruct((R, C), jnp.float32),
        scratch_types=[
            pltpu.VMEM((rpw, C), jnp.int32),
            pltpu.VMEM((rpw, C), jnp.float32),
            pltpu.SemaphoreType.DMA,
        ],
    )
    def k(w_hbm, idx_hbm, out_hbm, idx_v, out_v, sem):
        wid = _wid()
        base = wid * rpw
        pltpu.sync_copy(idx_hbm.at[pl.ds(base, rpw)], idx_v)

        def body(c, carry):
            pltpu.async_copy(w_hbm.at[idx_v.at[c]], out_v.at[c], sem).wait()
            return carry

        lax.fori_loop(0, rpw, body, 0)
        pltpu.sync_copy(out_v, out_hbm.at[pl.ds(base, rpw)])

    return k(w_bonds, idx2d)


def _sc_atom_gather(msg, w1d, idx1d, apply_relu, ch=256):
    TOT = idx1d.shape[0]
    AP = TOT // 32
    wpw = TOT // NW
    CH = ch
    G = CH // 32
    rpw = wpw // CH
    apw = rpw * G

    @functools.partial(
        pl.kernel,
        mesh=_mesh(),
        compiler_params=_SC_PARAMS,
        out_type=jax.ShapeDtypeStruct((AP, H), jnp.float32),
        scratch_types=[
            pltpu.VMEM((wpw,), jnp.int32),
            pltpu.VMEM((wpw,), jnp.float32),
            pltpu.VMEM((2, CH, H), jnp.float32),
            pltpu.VMEM((apw, H), jnp.float32),
            pltpu.SemaphoreType.DMA,
        ],
    )
    def k(msg_hbm, w_hbm, idx_hbm, out_hbm, idx_v, w_v, rows_v, out_v, sem):
        wid = _wid()
        pltpu.sync_copy(idx_hbm.at[pl.ds(wid * wpw, wpw)], idx_v)
        pltpu.sync_copy(w_hbm.at[pl.ds(wid * wpw, wpw)], w_v)

        pltpu.async_copy(
            msg_hbm.at[idx_v.at[pl.ds(0, CH)]], rows_v.at[0], sem)

        def do_chunk(c, b):
            @pl.when(c + 1 < rpw)
            def _():
                pltpu.async_copy(
                    msg_hbm.at[idx_v.at[pl.ds((c + 1) * CH, CH)]],
                    rows_v.at[1 - b], sem)

            pltpu.make_async_copy(
                msg_hbm.at[idx_v.at[pl.ds(c * CH, CH)]],
                rows_v.at[b], sem).wait()
            rb = rows_v.at[b]
            for g in range(G):
                zeros = tuple(
                    jnp.zeros((L,), jnp.float32) for _ in range(H // L))

                @plsc.parallel_loop(0, 32, unroll=4, carry=zeros)
                def accs(kk, accs_in):
                    r = g * 32 + kk
                    wv = plsc.load_gather(w_v, [_full16(c * CH + r)])
                    new = []
                    for j in range(H // L):
                        x = rb[r, pl.ds(j * L, L)]
                        if apply_relu:
                            x = jnp.maximum(x, 0.0)
                        new.append(accs_in[j] + wv * x)
                    return tuple(new)

                for j in range(H // L):
                    out_v[c * G + g, pl.ds(j * L, L)] = accs[j]

        def pair(t, carry):
            do_chunk(2 * t, 0)
            do_chunk(2 * t + 1, 1)
            return carry

        lax.fori_loop(0, rpw // 2, pair, 0)
        pltpu.sync_copy(out_v, out_hbm.at[pl.ds(wid * apw, apw)])

    return k(msg, w1d, idx1d)


def _sc_bond_update(amh, msgh, inp, w_bonds, b2a, b2revb, kb=88):
    NB = b2a.shape[0]
    bpw = NB // NW
    K = kb
    full = bpw // K
    tail = bpw - full * K

    @functools.partial(
        pl.kernel,
        mesh=_mesh(),
        compiler_params=_SC_PARAMS,
        out_type=jax.ShapeDtypeStruct((NB, H), jnp.float32),
        scratch_types=[
            pltpu.VMEM((bpw,), jnp.int32),
            pltpu.VMEM((bpw,), jnp.int32),
            pltpu.VMEM((bpw,), jnp.float32),
            pltpu.VMEM((2, K, H), jnp.float32),
            pltpu.VMEM((2, K, H), jnp.float32),
            pltpu.VMEM((2, K, H), jnp.float32),
            pltpu.VMEM((2, K, H), jnp.float32),
            pltpu.SemaphoreType.DMA,
            pltpu.SemaphoreType.DMA,
        ],
    )
    def k(am_hbm, msg_hbm, inp_hbm, w_hbm, ba_hbm, br_hbm, out_hbm,
          idxa, idxr, w_v, am_v, rev_v, i_v, o_v, semg, semw):
        wid = _wid()
        eb = wid * bpw
        pltpu.sync_copy(ba_hbm.at[pl.ds(eb, bpw)], idxa)
        pltpu.sync_copy(br_hbm.at[pl.ds(eb, bpw)], idxr)
        pltpu.sync_copy(w_hbm.at[pl.ds(eb, bpw)], w_v)

        def _issue(c, b, n):
            pltpu.async_copy(
                am_hbm.at[idxa.at[pl.ds(c * K, n)]],
                am_v.at[b].at[pl.ds(0, n)], semg)
            pltpu.async_copy(
                msg_hbm.at[idxr.at[pl.ds(c * K, n)]],
                rev_v.at[b].at[pl.ds(0, n)], semg)
            pltpu.async_copy(
                inp_hbm.at[pl.ds(eb + c * K, n)],
                i_v.at[b].at[pl.ds(0, n)], semg)

        def _wait(c, b, n):
            pltpu.make_async_copy(
                am_hbm.at[idxa.at[pl.ds(c * K, n)]],
                am_v.at[b].at[pl.ds(0, n)], semg).wait()
            pltpu.make_async_copy(
                msg_hbm.at[idxr.at[pl.ds(c * K, n)]],
                rev_v.at[b].at[pl.ds(0, n)], semg).wait()
            pltpu.make_async_copy(
                inp_hbm.at[pl.ds(eb + c * K, n)],
                i_v.at[b].at[pl.ds(0, n)], semg).wait()

        _issue(0, 0, K)

        def compute(c, b, n):
            amb, revb, ib = am_v.at[b], rev_v.at[b], i_v.at[b]
            ob = o_v.at[b]

            @plsc.parallel_loop(0, n, unroll=8)
            def _(r):
                wv = plsc.load_gather(w_v, [_full16(c * K + r)])
                for j in range(H // L):
                    s = pl.ds(j * L, L)
                    y = ib[r, s] + amb[r, s] - wv * revb[r, s]
                    ob[r, s] = jnp.maximum(y, 0.0)

        def do_chunk(c, b):
            @pl.when(c + 1 < full)
            def _():
                _issue(c + 1, 1 - b, K)

            _wait(c, b, K)

            @pl.when(c >= 2)
            def _():
                pltpu.make_async_copy(
                    o_v.at[b], out_hbm.at[pl.ds(eb + (c - 2) * K, K)],
                    semw).wait()

            compute(c, b, K)
            pltpu.async_copy(o_v.at[b], out_hbm.at[pl.ds(eb + c * K, K)], semw)

        def pair(t, carry):
            do_chunk(2 * t, 0)
            do_chunk(2 * t + 1, 1)
            return carry

        lax.fori_loop(0, full // 2, pair, 0)
        if full % 2:
            do_chunk(full - 1, (full - 1) % 2)
        for t in (full - 2, full - 1):
            pltpu.make_async_copy(
                o_v.at[t % 2], out_hbm.at[pl.ds(eb + t * K, K)], semw).wait()
        if tail:
            b3 = full % 2
            _issue(full, b3, tail)
            _wait(full, b3, tail)
            compute(full, b3, tail)
            pltpu.sync_copy(
                o_v.at[b3].at[pl.ds(0, tail)],
                out_hbm.at[pl.ds(eb + full * K, tail)])

    return k(amh, msgh, inp, w_bonds, b2a, b2revb)


def _mm_init(x, W, bk=2560):
    n = x.shape[0]

    def body(x_ref, w_ref, o_ref):
        o_ref[...] = jnp.dot(x_ref[...], w_ref[...],
                             preferred_element_type=jnp.float32)

    return pl.pallas_call(
        body,
        grid=(n // bk,),
        in_specs=[
            pl.BlockSpec((bk, H), lambda i: (i, 0)),
            pl.BlockSpec((H, H), lambda i: (0, 0)),
        ],
        out_specs=pl.BlockSpec((bk, H), lambda i: (i, 0)),
        out_shape=jax.ShapeDtypeStruct((n, H), jnp.float32),
    )(x, W)


def _mm_h(x, W, input_relu, bk=2560):
    n = x.shape[0]

    def body(x_ref, w_ref, o_ref):
        x = x_ref[...]
        if input_relu:
            x = jnp.maximum(x, 0.0)
        o_ref[...] = jnp.dot(x, w_ref[...], preferred_element_type=jnp.float32)

    return pl.pallas_call(
        body,
        grid=(n // bk,),
        in_specs=[
            pl.BlockSpec((bk, H), lambda i: (i, 0)),
            pl.BlockSpec((H, H), lambda i: (0, 0)),
        ],
        out_specs=pl.BlockSpec((bk, H), lambda i: (i, 0)),
        out_shape=jax.ShapeDtypeStruct((n, H), jnp.float32),
    )(x, W)


def _mm_final(f_atoms_p, am_p, batch3d, Wo_a, Wo_m, b_o2d, bk=1024):
    n = f_atoms_p.shape[0]
    nblk = n // bk

    def body(fa_ref, am_ref, b_ref, woa_ref, wom_ref, bo_ref, o_ref,
             acc_ref, cnt_ref):
        i = pl.program_id(0)

        @pl.when(i == 0)
        def _():
            acc_ref[...] = jnp.zeros_like(acc_ref)
            cnt_ref[...] = jnp.zeros_like(cnt_ref)

        h = jnp.dot(fa_ref[...], woa_ref[...], preferred_element_type=jnp.float32)
        h = h + jnp.dot(am_ref[...], wom_ref[...], preferred_element_type=jnp.float32)
        h = jnp.maximum(h + bo_ref[...], 0.0)
        ids = b_ref[0, 0, :]
        mol = lax.broadcasted_iota(jnp.int32, (NM, bk), 0)
        onehot = (ids[None, :] == mol).astype(jnp.float32)
        acc_ref[...] += jnp.dot(onehot, h, preferred_element_type=jnp.float32)
        cnt_ref[...] += jnp.sum(onehot, axis=1, keepdims=True)

        @pl.when(i == nblk - 1)
        def _():
            o_ref[...] = acc_ref[...] / jnp.maximum(cnt_ref[...], 1.0)

    return pl.pallas_call(
        body,
        grid=(nblk,),
        in_specs=[
            pl.BlockSpec((bk, H), lambda i: (i, 0)),
            pl.BlockSpec((bk, H), lambda i: (i, 0)),
            pl.BlockSpec((1, 1, bk), lambda i: (i, 0, 0)),
            pl.BlockSpec((H, H), lambda i: (0, 0)),
            pl.BlockSpec((H, H), lambda i: (0, 0)),
            pl.BlockSpec((1, H), lambda i: (0, 0)),
        ],
        out_specs=pl.BlockSpec((NM, H), lambda i: (0, 0)),
        out_shape=jax.ShapeDtypeStruct((NM, H), jnp.float32),
        scratch_shapes=[
            pltpu.VMEM((NM, H), jnp.float32),
            pltpu.VMEM((NM, 1), jnp.float32),
        ],
    )(f_atoms_p, am_p, batch3d, Wo_a, Wo_m, b_o2d)


def kernel(f_atoms, f_bonds, w_bonds, a2b, b2a, b2revb, batch,
           W_i, W_h, W_o, b_o):
    n_atoms = f_atoms.shape[0]
    nb = f_bonds.shape[0]
    maxnb = a2b.shape[1]

    a2b = a2b.astype(jnp.int32)
    b2a = b2a.astype(jnp.int32)
    b2revb = b2revb.astype(jnp.int32)

    ap = -(-n_atoms // 1024) * 1024
    a2b_p = jnp.pad(a2b, ((0, ap - n_atoms), (0, 0)))
    idx2d = a2b_p.reshape(ap * maxnb // 128, 128)
    idx1d = a2b_p.reshape(-1)

    w1d = _sc_wgather(w_bonds, idx2d).reshape(-1)

    inp = _mm_init(f_bonds, W_i)

    msgh = _mm_h(inp, W_h, True)
    am = _sc_atom_gather(inp, w1d, idx1d, True)
    amh = _mm_h(am, W_h, False, bk=1024)
    msg = _sc_bond_update(amh, msgh, inp, w_bonds, b2a, b2revb)

    msgh = _mm_h(msg, W_h, False)
    am = _sc_atom_gather(msg, w1d, idx1d, False)
    amh = _mm_h(am, W_h, False, bk=1024)
    msg = _sc_bond_update(amh, msgh, inp, w_bonds, b2a, b2revb)

    am = _sc_atom_gather(msg, w1d, idx1d, False)
    f_atoms_p = jnp.pad(f_atoms, ((0, ap - n_atoms), (0, 0)))
    batch_p = jnp.pad(batch.astype(jnp.int32), (0, ap - n_atoms),
                      constant_values=-1)
    batch3d = batch_p.reshape(ap // 1024, 1, 1024)
    Wo_a = W_o[:f_atoms.shape[1], :]
    Wo_m = W_o[f_atoms.shape[1]:, :]
    b_o2d = b_o.reshape(1, H)

    return _mm_final(f_atoms_p, am, batch3d, Wo_a, Wo_m, b_o2d)
